# baseline (speedup 1.0000x reference)
import functools

import jax
import jax.numpy as jnp
from jax import lax
from jax.experimental import pallas as pl
from jax.experimental.pallas import tpu as pltpu
from jax.experimental.pallas import tpu_sc as plsc

NC, NS, LANES = 2, 16, 16
NW = NC * NS
NB = 4096
NL = 200
D = 32
BB = 128
N_ITEMS = NL * (NB // BB)
IPW = N_ITEMS // NW
NTF = D // 8


def _build_sc_kernel():
  mesh = plsc.VectorSubcoreMesh(core_axis_name="c", subcore_axis_name="s")

  @functools.partial(
      pl.kernel,
      out_type=jax.ShapeDtypeStruct((NL, NTF, NB // BB, 8, BB),
                                    jnp.float32),
      mesh=mesh,
      compiler_params=pltpu.CompilerParams(
          needs_layout_passes=False, use_tc_tiling_on_sc=False),
      scratch_types=[
          pltpu.VMEM((BB,), jnp.int32),
          pltpu.VMEM((BB,), jnp.int32),
          pltpu.VMEM((BB, D), jnp.float32),
          pltpu.VMEM((BB, D), jnp.float32),
          pltpu.VMEM((D, BB), jnp.float32),
          pltpu.VMEM((D, BB), jnp.float32),
          pltpu.SemaphoreType.DMA,
          pltpu.SemaphoreType.DMA,
          pltpu.SemaphoreType.DMA,
          pltpu.SemaphoreType.DMA,
          pltpu.SemaphoreType.DMA,
          pltpu.SemaphoreType.DMA,
      ],
  )
  def k(xp_hbm, table_hbm, out_hbm, idx0, idx1, rows0, rows1, tb0, tb1,
        i0, i1, g0, g1, o0, o1):
    wid = lax.axis_index("s") * NC + lax.axis_index("c")
    item0 = wid * IPW

    def item_lt(i):
      g = item0 + i
      return lax.shift_right_logical(g, 5), lax.bitwise_and(g, 31)

    def idx_copy(i, idx_b, isem):
      l, tb = item_lt(i)
      off = pl.multiple_of(lax.shift_left(tb, 7), BB)
      return pltpu.make_async_copy(
          xp_hbm.at[l, pl.ds(off, BB)], idx_b, isem)

    def gather_copy(idx_b, rows_b, gsem):
      return pltpu.make_async_copy(table_hbm.at[idx_b], rows_b, gsem)

    def out_copies(i, tbuf, osem):
      l, tb = item_lt(i)
      return [
          pltpu.make_async_copy(
              tbuf.at[pl.ds(tf * 8, 8)], out_hbm.at[l, tf, tb], osem)
          for tf in range(NTF)
      ]

    def transpose_mask(idx_b, rows_b, tbuf):
      for r in range(BB // LANES):
        iv = idx_b[pl.ds(r * LANES, LANES)]
        mv = jnp.where(iv > 0, 1.0, 0.0).astype(jnp.float32)
        bidx = jax.lax.iota(jnp.int32, LANES) + (r * LANES)
        for f in range(D):
          col = plsc.load_gather(
              rows_b, [bidx, jnp.full((LANES,), f, jnp.int32)])
          tbuf[f, pl.ds(r * LANES, LANES)] = col * mv

    def half(i, idx_a, rows_a, tb_a, isem_a, gsem_a, osem_a,
             idx_b, rows_b, tb_b, isem_b, gsem_b, osem_b):
      @pl.when(i >= 2)
      def _wo():
        for c in out_copies(i - 2, tb_a, osem_a):
          c.wait()

      @pl.when(i + 1 < IPW)
      def _g():
        idx_copy(i + 1, idx_b, isem_b).wait()
        gather_copy(idx_b, rows_b, gsem_b).start()

      gather_copy(idx_a, rows_a, gsem_a).wait()

      @pl.when(i + 2 < IPW)
      def _pf():
        idx_copy(i + 2, idx_a, isem_a).start()

      transpose_mask(idx_a, rows_a, tb_a)
      for c in out_copies(i, tb_a, osem_a):
        c.start()

    idx_copy(0, idx0, i0).start()
    idx_copy(0, idx0, i0).wait()
    gather_copy(idx0, rows0, g0).start()
    idx_copy(1, idx1, i1).start()

    def body(kk, carry):
      i = kk * 2
      half(i, idx0, rows0, tb0, i0, g0, o0, idx1, rows1, tb1, i1, g1, o1)
      half(i + 1, idx1, rows1, tb1, i1, g1, o1,
           idx0, rows0, tb0, i0, g0, o0)
      return carry

    lax.fori_loop(0, IPW // 2, body, 0)
    for c in out_copies(IPW - 2, tb0, o0):
      c.wait()
    for c in out_copies(IPW - 1, tb1, o1):
      c.wait()

  return k


_gather = _build_sc_kernel()


def kernel(x, W):
  xp = x.transpose(1, 2, 0).reshape(NL, NB)
  out5 = _gather(xp, W)
  return out5.transpose(2, 4, 0, 1, 3).reshape(NB, NL, D)


# speedup vs baseline: 1.2309x; 1.2309x over previous
import functools

import jax
import jax.numpy as jnp
from jax import lax
from jax.experimental import pallas as pl
from jax.experimental.pallas import tpu as pltpu
from jax.experimental.pallas import tpu_sc as plsc

NC, NS, LANES = 2, 16, 16
NW = NC * NS
NB = 4096
NL = 200
D = 32
BB = 128
N_ITEMS = NL * (NB // BB)
IPW = N_ITEMS // NW
NTF = D // 8


def _build_sc_kernel():
  mesh = plsc.VectorSubcoreMesh(core_axis_name="c", subcore_axis_name="s")

  @functools.partial(
      pl.kernel,
      out_type=jax.ShapeDtypeStruct((NL, NTF, NB // BB, 8, BB),
                                    jnp.float32),
      mesh=mesh,
      compiler_params=pltpu.CompilerParams(
          needs_layout_passes=False, use_tc_tiling_on_sc=False),
      scratch_types=[
          pltpu.VMEM((BB,), jnp.int32),
          pltpu.VMEM((BB,), jnp.int32),
          pltpu.VMEM((BB, D), jnp.float32),
          pltpu.VMEM((BB, D), jnp.float32),
          pltpu.VMEM((D, BB), jnp.float32),
          pltpu.VMEM((D, BB), jnp.float32),
          pltpu.SemaphoreType.DMA,
          pltpu.SemaphoreType.DMA,
          pltpu.SemaphoreType.DMA,
          pltpu.SemaphoreType.DMA,
          pltpu.SemaphoreType.DMA,
          pltpu.SemaphoreType.DMA,
      ],
  )
  def k(xp_hbm, table_hbm, out_hbm, idx0, idx1, rows0, rows1, tb0, tb1,
        i0, i1, g0, g1, o0, o1):
    wid = lax.axis_index("s") * NC + lax.axis_index("c")
    item0 = wid * IPW

    def item_lt(i):
      g = item0 + i
      return lax.shift_right_logical(g, 5), lax.bitwise_and(g, 31)

    def idx_copy(i, idx_b, isem):
      l, tb = item_lt(i)
      off = pl.multiple_of(lax.shift_left(tb, 7), BB)
      return pltpu.make_async_copy(
          xp_hbm.at[l, pl.ds(off, BB)], idx_b, isem)

    def gather_copy(idx_b, rows_b, gsem):
      return pltpu.make_async_copy(table_hbm.at[idx_b], rows_b, gsem)

    def out_copies(i, tbuf, osem):
      l, tb = item_lt(i)
      return [
          pltpu.make_async_copy(
              tbuf.at[pl.ds(tf * 8, 8)], out_hbm.at[l, tf, tb], osem)
          for tf in range(NTF)
      ]

    def scan_fix(idx_b, rows_b):
      acc = idx_b[pl.ds(0, LANES)] <= 0
      for r in range(1, BB // LANES):
        acc = jnp.logical_or(acc, idx_b[pl.ds(r * LANES, LANES)] <= 0)
      nbad = plsc.all_reduce_population_count(acc)[0]

      @pl.when(nbad > 0)
      def _fix():
        for r in range(BB // LANES):
          v = idx_b[pl.ds(r * LANES, LANES)]
          gbad = plsc.all_reduce_population_count(v <= 0)[0]

          @pl.when(gbad > 0)
          def _fix_group():
            for l in range(LANES):
              m = (v[l] > 0).astype(jnp.float32)
              b = r * LANES + l
              rows_b[b, pl.ds(0, LANES)] = rows_b[b, pl.ds(0, LANES)] * m
              rows_b[b, pl.ds(LANES, LANES)] = (
                  rows_b[b, pl.ds(LANES, LANES)] * m)

    def transpose(rows_b, tbuf):
      f_lo = jax.lax.iota(jnp.int32, LANES)
      f_hi = f_lo + LANES
      for b in range(BB):
        bsp = jnp.full((LANES,), b, jnp.int32)
        plsc.store_scatter(tbuf, [f_lo, bsp], rows_b[b, pl.ds(0, LANES)])
        plsc.store_scatter(tbuf, [f_hi, bsp],
                           rows_b[b, pl.ds(LANES, LANES)])

    def half(i, idx_a, rows_a, tb_a, isem_a, gsem_a, osem_a,
             idx_b, rows_b, tb_b, isem_b, gsem_b, osem_b):
      @pl.when(i >= 2)
      def _wo():
        for c in out_copies(i - 2, tb_a, osem_a):
          c.wait()

      @pl.when(i + 1 < IPW)
      def _g():
        idx_copy(i + 1, idx_b, isem_b).wait()
        gather_copy(idx_b, rows_b, gsem_b).start()

      gather_copy(idx_a, rows_a, gsem_a).wait()
      scan_fix(idx_a, rows_a)

      @pl.when(i + 2 < IPW)
      def _pf():
        idx_copy(i + 2, idx_a, isem_a).start()

      transpose(rows_a, tb_a)
      for c in out_copies(i, tb_a, osem_a):
        c.start()

    idx_copy(0, idx0, i0).start()
    idx_copy(0, idx0, i0).wait()
    gather_copy(idx0, rows0, g0).start()
    idx_copy(1, idx1, i1).start()

    def body(kk, carry):
      i = kk * 2
      half(i, idx0, rows0, tb0, i0, g0, o0, idx1, rows1, tb1, i1, g1, o1)
      half(i + 1, idx1, rows1, tb1, i1, g1, o1,
           idx0, rows0, tb0, i0, g0, o0)
      return carry

    lax.fori_loop(0, IPW // 2, body, 0)
    for c in out_copies(IPW - 2, tb0, o0):
      c.wait()
    for c in out_copies(IPW - 1, tb1, o1):
      c.wait()

  return k


_gather = _build_sc_kernel()


def kernel(x, W):
  xp = x.transpose(1, 2, 0).reshape(NL, NB)
  out5 = _gather(xp, W)
  return out5.transpose(2, 4, 0, 1, 3).reshape(NB, NL, D)


# speedup vs baseline: 1.2549x; 1.0196x over previous
import functools

import jax
import jax.numpy as jnp
from jax import lax
from jax.experimental import pallas as pl
from jax.experimental.pallas import tpu as pltpu
from jax.experimental.pallas import tpu_sc as plsc

NC, NS, LANES = 2, 16, 16
NW = NC * NS
NB = 4096
NL = 200
D = 32
BB = 512
SUB = 128
NSUB = BB // SUB
NBLK = NB // BB
N_ITEMS = NL * NBLK
IPW = N_ITEMS // NW
NTF = D // 8
TBD = BB // SUB


def _build_sc_kernel():
  mesh = plsc.VectorSubcoreMesh(core_axis_name="c", subcore_axis_name="s")

  @functools.partial(
      pl.kernel,
      out_type=jax.ShapeDtypeStruct((NL, NTF, NB // SUB, 8, SUB),
                                    jnp.float32),
      mesh=mesh,
      compiler_params=pltpu.CompilerParams(
          needs_layout_passes=False, use_tc_tiling_on_sc=False),
      scratch_types=[
          pltpu.VMEM((NSUB, SUB), jnp.int32),
          pltpu.VMEM((NSUB, SUB), jnp.int32),
          pltpu.VMEM((BB, D), jnp.float32),
          pltpu.VMEM((BB, D), jnp.float32),
          pltpu.VMEM((NTF, TBD, 8, SUB), jnp.float32),
          pltpu.VMEM((NTF, TBD, 8, SUB), jnp.float32),
          pltpu.SemaphoreType.DMA,
          pltpu.SemaphoreType.DMA,
          pltpu.SemaphoreType.DMA,
          pltpu.SemaphoreType.DMA,
          pltpu.SemaphoreType.DMA,
          pltpu.SemaphoreType.DMA,
      ],
  )
  def k(xp_hbm, table_hbm, out_hbm, idx0, idx1, rows0, rows1, tb0, tb1,
        i0, i1, g0, g1, o0, o1):
    wid = lax.axis_index("s") * NC + lax.axis_index("c")
    item0 = wid * IPW

    def item_parts(i):
      g = item0 + i
      return lax.shift_right_logical(g, 3), lax.bitwise_and(g, NBLK - 1)

    def idx_copy(i, idx_b, isem):
      l, ib = item_parts(i)
      off = pl.multiple_of(lax.shift_left(ib, 2), NSUB)
      return pltpu.make_async_copy(
          xp_hbm.at[l, pl.ds(off, NSUB)], idx_b, isem)

    def gather_copies(idx_b, rows_b, gsem):
      return [
          pltpu.make_async_copy(
              table_hbm.at[idx_b.at[j]],
              rows_b.at[pl.ds(j * SUB, SUB)], gsem)
          for j in range(NSUB)
      ]

    def out_copies(i, tbuf, osem):
      l, ib = item_parts(i)
      tb0_ = pl.multiple_of(lax.shift_left(ib, 2), TBD)
      return [
          pltpu.make_async_copy(
              tbuf.at[tf], out_hbm.at[l, tf, pl.ds(tb0_, TBD)], osem)
          for tf in range(NTF)
      ]

    def scan_fix(idx_b, rows_b):
      acc = idx_b[0, pl.ds(0, LANES)] <= 0
      for t in range(1, BB // LANES):
        j, r = divmod(t, SUB // LANES)
        acc = jnp.logical_or(acc, idx_b[j, pl.ds(r * LANES, LANES)] <= 0)
      nbad = plsc.all_reduce_population_count(acc)[0]

      @pl.when(nbad > 0)
      def _fix():
        for j in range(NSUB):
          def grp_body(r, _, j=j):
            v = idx_b[j, pl.ds(r * LANES, LANES)]
            gbad = plsc.all_reduce_population_count(v <= 0)[0]

            @pl.when(gbad > 0)
            def _fix_group():
              for l in range(LANES):
                m = (v[l] > 0).astype(jnp.float32)
                b = j * SUB + r * LANES + l
                rows_b[b, pl.ds(0, LANES)] = (
                    rows_b[b, pl.ds(0, LANES)] * m)
                rows_b[b, pl.ds(LANES, LANES)] = (
                    rows_b[b, pl.ds(LANES, LANES)] * m)

            return 0
          lax.fori_loop(0, SUB // LANES, grp_body, 0)

    def transpose(rows_b, tbuf):
      it = jax.lax.iota(jnp.int32, LANES)
      tf_lo = lax.shift_right_logical(it, 3)
      fi_v = lax.bitwise_and(it, 7)
      tf_hi = tf_lo + 2

      def grp(gq, _):
        b0 = lax.shift_left(gq, 4)
        tb = lax.shift_right_logical(b0, 7)
        tbsp = jnp.full((LANES,), 0, jnp.int32) + tb
        for l in range(LANES):
          b = b0 + l
          bisp = jnp.full((LANES,), 0, jnp.int32) + lax.bitwise_and(b, 127)
          plsc.store_scatter(tbuf, [tf_lo, tbsp, fi_v, bisp],
                             rows_b[b, pl.ds(0, LANES)])
          plsc.store_scatter(tbuf, [tf_hi, tbsp, fi_v, bisp],
                             rows_b[b, pl.ds(LANES, LANES)])
        return 0

      lax.fori_loop(0, BB // LANES, grp, 0)

    def half(i, idx_a, rows_a, tb_a, isem_a, gsem_a, osem_a,
             idx_b, rows_b, tb_b, isem_b, gsem_b, osem_b):
      @pl.when(i >= 2)
      def _wo():
        for c in out_copies(i - 2, tb_a, osem_a):
          c.wait()

      @pl.when(i + 1 < IPW)
      def _g():
        idx_copy(i + 1, idx_b, isem_b).wait()
        for c in gather_copies(idx_b, rows_b, gsem_b):
          c.start()

      for c in gather_copies(idx_a, rows_a, gsem_a):
        c.wait()
      scan_fix(idx_a, rows_a)

      @pl.when(i + 2 < IPW)
      def _pf():
        idx_copy(i + 2, idx_a, isem_a).start()

      transpose(rows_a, tb_a)
      for c in out_copies(i, tb_a, osem_a):
        c.start()

    idx_copy(0, idx0, i0).start()
    idx_copy(0, idx0, i0).wait()
    for c in gather_copies(idx0, rows0, g0):
      c.start()
    idx_copy(1, idx1, i1).start()

    def body(kk, carry):
      i = kk * 2
      half(i, idx0, rows0, tb0, i0, g0, o0, idx1, rows1, tb1, i1, g1, o1)
      half(i + 1, idx1, rows1, tb1, i1, g1, o1,
           idx0, rows0, tb0, i0, g0, o0)
      return carry

    lax.fori_loop(0, IPW // 2, body, 0)
    for c in out_copies(IPW - 2, tb0, o0):
      c.wait()
    for c in out_copies(IPW - 1, tb1, o1):
      c.wait()

  return k


_gather = _build_sc_kernel()


def kernel(x, W):
  xp = x.transpose(1, 2, 0).reshape(NL, NB // SUB, SUB)
  out5 = _gather(xp, W)
  return out5.transpose(2, 4, 0, 1, 3).reshape(NB, NL, D)


# speedup vs baseline: 1.7527x; 1.3967x over previous
import functools

import jax
import jax.numpy as jnp
from jax import lax
from jax.experimental import pallas as pl
from jax.experimental.pallas import tpu as pltpu
from jax.experimental.pallas import tpu_sc as plsc

NC, NS, LANES = 2, 16, 16
NW = NC * NS
NB = 4096
NL = 200
D = 32
BB = 512
SUB = 128
NSUB = BB // SUB
NBLK = NB // BB
N_ITEMS = NL * NBLK
IPW = N_ITEMS // NW
NTF = D // 8
TBD = BB // SUB


def _build_sc_kernel():
  mesh = plsc.VectorSubcoreMesh(core_axis_name="c", subcore_axis_name="s")

  @functools.partial(
      pl.kernel,
      out_type=jax.ShapeDtypeStruct((NL, NTF, NB // SUB, 8, SUB),
                                    jnp.float32),
      mesh=mesh,
      compiler_params=pltpu.CompilerParams(
          needs_layout_passes=False, use_tc_tiling_on_sc=False),
      scratch_types=[
          pltpu.VMEM((NSUB, SUB), jnp.int32),
          pltpu.VMEM((NSUB, SUB), jnp.int32),
          pltpu.VMEM((BB, D), jnp.float32),
          pltpu.VMEM((BB, D), jnp.float32),
          pltpu.VMEM((NTF, TBD + 1, 8, SUB + 1), jnp.float32),
          pltpu.VMEM((NTF, TBD + 1, 8, SUB + 1), jnp.float32),
          pltpu.SemaphoreType.DMA,
          pltpu.SemaphoreType.DMA,
          pltpu.SemaphoreType.DMA,
          pltpu.SemaphoreType.DMA,
          pltpu.SemaphoreType.DMA,
          pltpu.SemaphoreType.DMA,
      ],
  )
  def k(xp_hbm, table_hbm, out_hbm, idx0, idx1, rows0, rows1, tb0, tb1,
        i0, i1, g0, g1, o0, o1):
    wid = lax.axis_index("s") * NC + lax.axis_index("c")
    item0 = wid * IPW

    def item_parts(i):
      g = item0 + i
      return lax.shift_right_logical(g, 3), lax.bitwise_and(g, NBLK - 1)

    def idx_copy(i, idx_b, isem):
      l, ib = item_parts(i)
      off = pl.multiple_of(lax.shift_left(ib, 2), NSUB)
      return pltpu.make_async_copy(
          xp_hbm.at[l, pl.ds(off, NSUB)], idx_b, isem)

    def gather_copies(idx_b, rows_b, gsem):
      return [
          pltpu.make_async_copy(
              table_hbm.at[idx_b.at[j]],
              rows_b.at[pl.ds(j * SUB, SUB)], gsem)
          for j in range(NSUB)
      ]

    def out_copies(i, tbuf, osem):
      l, ib = item_parts(i)
      tb0_ = pl.multiple_of(lax.shift_left(ib, 2), TBD)
      return [
          pltpu.make_async_copy(
              tbuf.at[tf, pl.ds(0, TBD), pl.ds(0, 8), pl.ds(0, SUB)],
              out_hbm.at[l, tf, pl.ds(tb0_, TBD)], osem)
          for tf in range(NTF)
      ]

    def scan_fix(idx_b, rows_b):
      acc = idx_b[0, pl.ds(0, LANES)] <= 0
      for t in range(1, BB // LANES):
        j, r = divmod(t, SUB // LANES)
        acc = jnp.logical_or(acc, idx_b[j, pl.ds(r * LANES, LANES)] <= 0)
      nbad = plsc.all_reduce_population_count(acc)[0]

      @pl.when(nbad > 0)
      def _fix():
        for j in range(NSUB):
          def grp_body(r, _, j=j):
            v = idx_b[j, pl.ds(r * LANES, LANES)]
            gbad = plsc.all_reduce_population_count(v <= 0)[0]

            @pl.when(gbad > 0)
            def _fix_group():
              for l in range(LANES):
                m = (v[l] > 0).astype(jnp.float32)
                b = j * SUB + r * LANES + l
                rows_b[b, pl.ds(0, LANES)] = (
                    rows_b[b, pl.ds(0, LANES)] * m)
                rows_b[b, pl.ds(LANES, LANES)] = (
                    rows_b[b, pl.ds(LANES, LANES)] * m)

            return 0
          lax.fori_loop(0, SUB // LANES, grp_body, 0)

    def transpose(rows_b, tbuf):
      it = jax.lax.iota(jnp.int32, LANES)
      tf_lo = lax.shift_right_logical(it, 3)
      fi_v = lax.bitwise_and(it, 7)
      tf_hi = tf_lo + 2

      def grp(gq, _):
        b0 = lax.shift_left(gq, 4)
        tb = lax.shift_right_logical(b0, 7)
        tbsp = jnp.full((LANES,), 0, jnp.int32) + tb
        for l in range(LANES):
          b = b0 + l
          bisp = jnp.full((LANES,), 0, jnp.int32) + lax.bitwise_and(b, 127)
          plsc.store_scatter(tbuf, [tf_lo, tbsp, fi_v, bisp],
                             rows_b[b, pl.ds(0, LANES)])
          plsc.store_scatter(tbuf, [tf_hi, tbsp, fi_v, bisp],
                             rows_b[b, pl.ds(LANES, LANES)])
        return 0

      lax.fori_loop(0, BB // LANES, grp, 0)

    def half(i, idx_a, rows_a, tb_a, isem_a, gsem_a, osem_a,
             idx_b, rows_b, tb_b, isem_b, gsem_b, osem_b):
      @pl.when(i >= 2)
      def _wo():
        for c in out_copies(i - 2, tb_a, osem_a):
          c.wait()

      @pl.when(i + 1 < IPW)
      def _g():
        idx_copy(i + 1, idx_b, isem_b).wait()
        for c in gather_copies(idx_b, rows_b, gsem_b):
          c.start()

      for c in gather_copies(idx_a, rows_a, gsem_a):
        c.wait()
      scan_fix(idx_a, rows_a)

      @pl.when(i + 2 < IPW)
      def _pf():
        idx_copy(i + 2, idx_a, isem_a).start()

      transpose(rows_a, tb_a)
      for c in out_copies(i, tb_a, osem_a):
        c.start()

    idx_copy(0, idx0, i0).start()
    idx_copy(0, idx0, i0).wait()
    for c in gather_copies(idx0, rows0, g0):
      c.start()
    idx_copy(1, idx1, i1).start()

    def body(kk, carry):
      i = kk * 2
      half(i, idx0, rows0, tb0, i0, g0, o0, idx1, rows1, tb1, i1, g1, o1)
      half(i + 1, idx1, rows1, tb1, i1, g1, o1,
           idx0, rows0, tb0, i0, g0, o0)
      return carry

    lax.fori_loop(0, IPW // 2, body, 0)
    for c in out_copies(IPW - 2, tb0, o0):
      c.wait()
    for c in out_copies(IPW - 1, tb1, o1):
      c.wait()

  return k


_gather = _build_sc_kernel()


def kernel(x, W):
  xp = x.transpose(1, 2, 0).reshape(NL, NB // SUB, SUB)
  out5 = _gather(xp, W)
  return out5.transpose(2, 4, 0, 1, 3).reshape(NB, NL, D)
